# baseline (speedup 1.0000x reference)
import jax
import jax.numpy as jnp
from jax.experimental import pallas as pl
from jax.experimental.pallas import tpu as pltpu

_BATCH = 16384
_STATE_DIM = 256
_HIDDEN = 1024
_NUM_ACTIONS = 257
_PAD = 384
_ROWS = 512


def _fused_body(s_ref, w1_ref, b1_ref, w2t_ref, b2_ref, o_ref,
                w1f_ref, w2tf_ref, w1bf_ref, w2tbf_ref, b2p_ref, sem):
    @pl.when(pl.program_id(0) == 0)
    def _prep_weights():
        c1 = pltpu.make_async_copy(w1_ref, w1f_ref, sem)
        c1.start()
        c2 = pltpu.make_async_copy(w2t_ref, w2tf_ref, sem)
        c2.start()
        c1.wait()
        c2.wait()
        w1bf_ref[...] = w1f_ref[...].astype(jnp.bfloat16)
        w2tbf_ref[...] = jnp.zeros((_PAD, _HIDDEN), jnp.bfloat16)
        w2tbf_ref[:_NUM_ACTIONS, :] = w2tf_ref[...].astype(jnp.bfloat16)
        b2p_ref[...] = jnp.full((1, _PAD), -1e9, jnp.float32)
        b2p_ref[:, :_NUM_ACTIONS] = b2_ref[...]

    s = s_ref[...]
    h = jnp.dot(s.astype(jnp.bfloat16), w1bf_ref[...],
                preferred_element_type=jnp.float32)
    h = jnp.maximum(h + b1_ref[...], 0.0).astype(jnp.bfloat16)
    logits = jax.lax.dot_general(
        h, w2tbf_ref[...], (((1,), (1,)), ((), ())),
        preferred_element_type=jnp.float32)
    logits = logits + b2p_ref[...]
    mx = jnp.max(logits, axis=1, keepdims=True)
    e = jnp.exp(logits - mx)
    cont = (s < 2.0).astype(jnp.float32)
    col = jax.lax.broadcasted_iota(jnp.int32, (s.shape[0], 128), 1)
    tail = (col == 0).astype(jnp.float32)
    mask = jnp.concatenate([cont, tail], axis=1)
    me = e * mask
    out = me / jnp.sum(me, axis=1, keepdims=True)
    o_ref[...] = out.T[:_NUM_ACTIONS, :]


def kernel(states, W1, b1, W2, b2):
    grid = (_BATCH // _ROWS,)
    _hbm = lambda x: pltpu.with_memory_space_constraint(
        x, pltpu.MemorySpace.HBM)
    out_t = pl.pallas_call(
        _fused_body,
        grid=grid,
        in_specs=[
            pl.BlockSpec((_ROWS, _STATE_DIM), lambda i: (i, 0)),
            pl.BlockSpec(memory_space=pl.ANY),
            pl.BlockSpec((1, _HIDDEN), lambda i: (0, 0)),
            pl.BlockSpec(memory_space=pl.ANY),
            pl.BlockSpec((1, _NUM_ACTIONS), lambda i: (0, 0)),
        ],
        out_specs=pl.BlockSpec((_NUM_ACTIONS, _ROWS), lambda i: (0, i)),
        out_shape=jax.ShapeDtypeStruct((_NUM_ACTIONS, _BATCH), jnp.float32),
        scratch_shapes=[
            pltpu.VMEM((_STATE_DIM, _HIDDEN), jnp.float32),
            pltpu.VMEM((_NUM_ACTIONS, _HIDDEN), jnp.float32),
            pltpu.VMEM((_STATE_DIM, _HIDDEN), jnp.bfloat16),
            pltpu.VMEM((_PAD, _HIDDEN), jnp.bfloat16),
            pltpu.VMEM((1, _PAD), jnp.float32),
            pltpu.SemaphoreType.DMA,
        ],
        compiler_params=pltpu.CompilerParams(
            dimension_semantics=("arbitrary",),
        ),
    )(_hbm(states), _hbm(W1), _hbm(b1.reshape(1, _HIDDEN)), _hbm(W2.T),
      _hbm(b2.reshape(1, _NUM_ACTIONS)))
    return out_t.T


# speedup vs baseline: 1.0202x; 1.0202x over previous
import jax
import jax.numpy as jnp
from jax.experimental import pallas as pl
from jax.experimental.pallas import tpu as pltpu

_BATCH = 16384
_STATE_DIM = 256
_HIDDEN = 1024
_NUM_ACTIONS = 257
_PAD = 384
_ROWS = 512


def _fused_body(s_ref, w1_ref, b1_ref, w2t_ref, b2_ref, o_ref,
                w1f_ref, w2tf_ref, w1bf_ref, w2tbf_ref, b2p_ref, sem):
    @pl.when(pl.program_id(0) == 0)
    def _prep_weights():
        c1 = pltpu.make_async_copy(w1_ref, w1f_ref, sem)
        c1.start()
        c2 = pltpu.make_async_copy(w2t_ref, w2tf_ref, sem)
        c2.start()
        c1.wait()
        c2.wait()
        w1bf_ref[...] = w1f_ref[...].astype(jnp.bfloat16)
        w2tbf_ref[...] = jnp.zeros((_PAD, _HIDDEN), jnp.bfloat16)
        w2tbf_ref[:_NUM_ACTIONS, :] = w2tf_ref[...].astype(jnp.bfloat16)
        b2p_ref[...] = jnp.full((1, _PAD), -1e9, jnp.float32)
        b2p_ref[:, :_NUM_ACTIONS] = b2_ref[...]

    half = _ROWS // 2
    for i in range(2):
        rows = pl.ds(i * half, half)
        s = s_ref[rows, :]
        h = jnp.dot(s.astype(jnp.bfloat16), w1bf_ref[...],
                    preferred_element_type=jnp.float32)
        h = jnp.maximum(h + b1_ref[...], 0.0).astype(jnp.bfloat16)
        logits = jax.lax.dot_general(
            h, w2tbf_ref[...], (((1,), (1,)), ((), ())),
            preferred_element_type=jnp.float32)
        logits = logits + b2p_ref[...]
        mx = jnp.max(logits, axis=1, keepdims=True)
        e = jnp.exp(logits - mx)
        cont = (s < 2.0).astype(jnp.float32)
        col = jax.lax.broadcasted_iota(jnp.int32, (half, 128), 1)
        tail = (col == 0).astype(jnp.float32)
        mask = jnp.concatenate([cont, tail], axis=1)
        me = e * mask
        out = me / jnp.sum(me, axis=1, keepdims=True)
        o_ref[:, rows] = out.T[:_NUM_ACTIONS, :]


def kernel(states, W1, b1, W2, b2):
    grid = (_BATCH // _ROWS,)
    _hbm = lambda x: pltpu.with_memory_space_constraint(
        x, pltpu.MemorySpace.HBM)
    out_t = pl.pallas_call(
        _fused_body,
        grid=grid,
        in_specs=[
            pl.BlockSpec((_ROWS, _STATE_DIM), lambda i: (i, 0)),
            pl.BlockSpec(memory_space=pl.ANY),
            pl.BlockSpec((1, _HIDDEN), lambda i: (0, 0)),
            pl.BlockSpec(memory_space=pl.ANY),
            pl.BlockSpec((1, _NUM_ACTIONS), lambda i: (0, 0)),
        ],
        out_specs=pl.BlockSpec((_NUM_ACTIONS, _ROWS), lambda i: (0, i)),
        out_shape=jax.ShapeDtypeStruct((_NUM_ACTIONS, _BATCH), jnp.float32),
        scratch_shapes=[
            pltpu.VMEM((_STATE_DIM, _HIDDEN), jnp.float32),
            pltpu.VMEM((_NUM_ACTIONS, _HIDDEN), jnp.float32),
            pltpu.VMEM((_STATE_DIM, _HIDDEN), jnp.bfloat16),
            pltpu.VMEM((_PAD, _HIDDEN), jnp.bfloat16),
            pltpu.VMEM((1, _PAD), jnp.float32),
            pltpu.SemaphoreType.DMA,
        ],
        compiler_params=pltpu.CompilerParams(
            dimension_semantics=("arbitrary",),
        ),
    )(_hbm(states), _hbm(W1), _hbm(b1.reshape(1, _HIDDEN)), _hbm(W2.T),
      _hbm(b2.reshape(1, _NUM_ACTIONS)))
    return out_t.T
